# Initial kernel scaffold; baseline (speedup 1.0000x reference)
import jax
import jax.numpy as jnp
from jax.experimental import pallas as pl


def kernel(input, weight):
    raise NotImplementedError("write your pallas kernel here")


# speedup vs baseline: 2.3290x; 2.3290x over previous
import functools

import jax
import jax.numpy as jnp
from jax import lax
from jax.experimental import pallas as pl
from jax.experimental.pallas import tpu as pltpu
from jax.experimental.pallas import tpu_sc as plsc

NUM_EMB = 100000
DIM = 128
EPS = 1e-5
B = 4096 * 50
NC, NS = 2, 16
NW = NC * NS
BPW = B // NW
CHUNK = 128
NCHUNK = BPW // CHUNK

RED_BLK = 2000


def _absum_body(w_ref, acc_ref):
    @pl.when(pl.program_id(0) == 0)
    def _():
        acc_ref[...] = jnp.zeros_like(acc_ref)

    x = jnp.abs(w_ref[...])
    acc_ref[...] += jnp.sum(x.reshape(RED_BLK // 8, 8, DIM), axis=0)


def _absum(weight):
    return pl.pallas_call(
        _absum_body,
        grid=(NUM_EMB // RED_BLK,),
        in_specs=[pl.BlockSpec((RED_BLK, DIM), lambda i: (i, 0))],
        out_specs=pl.BlockSpec((8, DIM), lambda i: (0, 0)),
        out_shape=jax.ShapeDtypeStruct((8, DIM), jnp.float32),
    )(weight)


_MESH = plsc.VectorSubcoreMesh(core_axis_name="c", subcore_axis_name="s")


@functools.partial(
    pl.kernel,
    out_type=jax.ShapeDtypeStruct((B, DIM), jnp.float32),
    mesh=_MESH,
    scratch_types=[
        pltpu.VMEM((NCHUNK, CHUNK), jnp.int32),
        pltpu.VMEM((CHUNK, DIM), jnp.float32),
        pltpu.VMEM((16,), jnp.float32),
        pltpu.SemaphoreType.DMA,
    ],
)
def _sc_lookup(idx_hbm, scale_hbm, w_hbm, out_hbm, idx_v, rows_v, scale_v, gsem):
    wid = lax.axis_index("s") * NC + lax.axis_index("c")
    base = wid * BPW
    pltpu.sync_copy(scale_hbm, scale_v)
    pltpu.sync_copy(idx_hbm.at[wid], idx_v)
    scale = scale_v[...]

    def chunk_body(g, _):
        pltpu.async_copy(w_hbm.at[idx_v.at[g]], rows_v, gsem).wait()

        def row_body(r, _):
            for j in range(DIM // 16):
                w = rows_v[r, pl.ds(j * 16, 16)]
                rows_v[r, pl.ds(j * 16, 16)] = jnp.sign(w) * scale
            return 0

        lax.fori_loop(0, CHUNK, row_body, 0)
        pltpu.sync_copy(rows_v, out_hbm.at[pl.ds(base + g * CHUNK, CHUNK)])
        return 0

    lax.fori_loop(0, NCHUNK, chunk_body, 0)


def kernel(input, weight):
    acc = _absum(weight)
    scale = jnp.maximum(jnp.sum(acc) / (NUM_EMB * DIM), EPS)
    scale16 = jnp.full((16,), scale, jnp.float32)
    idx = input.reshape(NW, NCHUNK, CHUNK).astype(jnp.int32)
    out = _sc_lookup(idx, scale16, weight)
    return out.reshape(input.shape[0], input.shape[1], DIM)


# speedup vs baseline: 6.1063x; 2.6218x over previous
import functools

import jax
import jax.numpy as jnp
from jax import lax
from jax.experimental import pallas as pl
from jax.experimental.pallas import tpu as pltpu
from jax.experimental.pallas import tpu_sc as plsc

NUM_EMB = 100000
DIM = 128
EPS = 1e-5
B = 4096 * 50
NC, NS = 2, 16
NW = NC * NS
BPW = B // NW
CHUNK = 128
NCHUNK = BPW // CHUNK

RED_BLK = 2000


def _absum_body(w_ref, acc_ref):
    @pl.when(pl.program_id(0) == 0)
    def _():
        acc_ref[...] = jnp.zeros_like(acc_ref)

    x = jnp.abs(w_ref[...])
    acc_ref[...] += jnp.sum(x.reshape(RED_BLK // 8, 8, DIM), axis=0)


def _absum(weight):
    return pl.pallas_call(
        _absum_body,
        grid=(NUM_EMB // RED_BLK,),
        in_specs=[pl.BlockSpec((RED_BLK, DIM), lambda i: (i, 0))],
        out_specs=pl.BlockSpec((8, DIM), lambda i: (0, 0)),
        out_shape=jax.ShapeDtypeStruct((8, DIM), jnp.float32),
    )(weight)


_MESH = plsc.VectorSubcoreMesh(core_axis_name="c", subcore_axis_name="s")


@functools.partial(
    pl.kernel,
    out_type=jax.ShapeDtypeStruct((B, DIM), jnp.float32),
    mesh=_MESH,
    scratch_types=[
        pltpu.VMEM((NCHUNK, CHUNK), jnp.int32),
        pltpu.VMEM((CHUNK, DIM), jnp.float32),
        pltpu.VMEM((CHUNK, DIM), jnp.float32),
        pltpu.VMEM((16,), jnp.float32),
        pltpu.SemaphoreType.DMA,
        pltpu.SemaphoreType.DMA,
    ],
)
def _sc_lookup(idx_hbm, scale_hbm, w_hbm, out_hbm,
               idx_v, rows0, rows1, scale_v, gsem0, gsem1):
    wid = lax.axis_index("s") * NC + lax.axis_index("c")
    base = wid * BPW
    pltpu.sync_copy(scale_hbm, scale_v)
    pltpu.sync_copy(idx_hbm.at[wid], idx_v)
    scale = scale_v[...]

    def start_gather(g, rows, gsem):
        return pltpu.async_copy(w_hbm.at[idx_v.at[g]], rows, gsem)

    def quantize_buf(rows):
        def row_body(r, _):
            for j in range(DIM // 16):
                w = rows[r, pl.ds(j * 16, 16)]
                rows[r, pl.ds(j * 16, 16)] = jnp.sign(w) * scale
            return 0

        lax.fori_loop(0, CHUNK, row_body, 0)

    def drain(g, rows, gsem):
        pltpu.make_async_copy(w_hbm.at[idx_v.at[g]], rows, gsem).wait()
        quantize_buf(rows)
        pltpu.sync_copy(rows, out_hbm.at[pl.ds(base + g * CHUNK, CHUNK)])

    start_gather(0, rows0, gsem0)

    def pair_body(i, _):
        g = 2 * i
        start_gather(g + 1, rows1, gsem1)
        drain(g, rows0, gsem0)
        start_gather(g + 2, rows0, gsem0)
        drain(g + 1, rows1, gsem1)
        return 0

    lax.fori_loop(0, NCHUNK // 2 - 1, pair_body, 0)
    g = NCHUNK - 2
    start_gather(g + 1, rows1, gsem1)
    drain(g, rows0, gsem0)
    drain(g + 1, rows1, gsem1)


def kernel(input, weight):
    acc = _absum(weight)
    scale = jnp.maximum(jnp.sum(acc) / (NUM_EMB * DIM), EPS)
    scale16 = jnp.full((16,), scale, jnp.float32)
    idx = input.T.reshape(NW, NCHUNK, CHUNK).astype(jnp.int32)
    out = _sc_lookup(idx, scale16, weight)
    n, m = input.shape
    return out.reshape(m, n, DIM).transpose(1, 0, 2)


# speedup vs baseline: 6.6613x; 1.0909x over previous
import functools

import jax
import jax.numpy as jnp
from jax import lax
from jax.experimental import pallas as pl
from jax.experimental.pallas import tpu as pltpu
from jax.experimental.pallas import tpu_sc as plsc

NUM_EMB = 100000
DIM = 128
EPS = 1e-5
B = 4096 * 50
NC, NS = 2, 16
NW = NC * NS
BPW = B // NW
CHUNK = 128
NCHUNK = BPW // CHUNK

RED_BLK = 4000


def _scale_body(w_ref, acc_ref):
    @pl.when(pl.program_id(0) == 0)
    def _():
        acc_ref[...] = jnp.zeros_like(acc_ref)

    x = jnp.abs(w_ref[...])
    acc_ref[...] += jnp.sum(x.reshape(RED_BLK // 8, 8, DIM), axis=0)

    @pl.when(pl.program_id(0) == NUM_EMB // RED_BLK - 1)
    def _():
        s = jnp.maximum(jnp.sum(acc_ref[...]) / (NUM_EMB * DIM), EPS)
        acc_ref[...] = jnp.full((8, DIM), s, jnp.float32)


def _scale_bcast(weight):
    return pl.pallas_call(
        _scale_body,
        grid=(NUM_EMB // RED_BLK,),
        in_specs=[pl.BlockSpec((RED_BLK, DIM), lambda i: (i, 0))],
        out_specs=pl.BlockSpec((8, DIM), lambda i: (0, 0)),
        out_shape=jax.ShapeDtypeStruct((8, DIM), jnp.float32),
    )(weight)


_MESH = plsc.VectorSubcoreMesh(core_axis_name="c", subcore_axis_name="s")


@functools.partial(
    pl.kernel,
    out_type=jax.ShapeDtypeStruct((B, DIM), jnp.float32),
    mesh=_MESH,
    scratch_types=[
        pltpu.VMEM((NCHUNK, CHUNK), jnp.int32),
        pltpu.VMEM((CHUNK, DIM), jnp.float32),
        pltpu.VMEM((CHUNK, DIM), jnp.float32),
        pltpu.VMEM((8, DIM), jnp.float32),
        pltpu.SemaphoreType.DMA,
        pltpu.SemaphoreType.DMA,
    ],
)
def _sc_lookup(idx_hbm, scale_hbm, w_hbm, out_hbm,
               idx_v, rows0, rows1, scale_v, gsem0, gsem1):
    wid = lax.axis_index("s") * NC + lax.axis_index("c")
    base = wid * BPW
    pltpu.sync_copy(scale_hbm, scale_v)
    pltpu.sync_copy(idx_hbm.at[wid], idx_v)
    scale = scale_v[0, pl.ds(0, 16)]

    def start_gather(g, rows, gsem):
        return pltpu.async_copy(w_hbm.at[idx_v.at[g]], rows, gsem)

    def quantize_buf(rows):
        def row_body(r, _):
            for j in range(DIM // 16):
                w = rows[r, pl.ds(j * 16, 16)]
                rows[r, pl.ds(j * 16, 16)] = jnp.sign(w) * scale
            return 0

        lax.fori_loop(0, CHUNK, row_body, 0)

    def drain(g, rows, gsem):
        pltpu.make_async_copy(w_hbm.at[idx_v.at[g]], rows, gsem).wait()
        quantize_buf(rows)
        pltpu.sync_copy(rows, out_hbm.at[pl.ds(base + g * CHUNK, CHUNK)])

    start_gather(0, rows0, gsem0)

    def pair_body(i, _):
        g = 2 * i
        start_gather(g + 1, rows1, gsem1)
        drain(g, rows0, gsem0)
        start_gather(g + 2, rows0, gsem0)
        drain(g + 1, rows1, gsem1)
        return 0

    lax.fori_loop(0, NCHUNK // 2 - 1, pair_body, 0)
    g = NCHUNK - 2
    start_gather(g + 1, rows1, gsem1)
    drain(g, rows0, gsem0)
    drain(g + 1, rows1, gsem1)


def kernel(input, weight):
    scale_bcast = _scale_bcast(weight)
    idx = input.T.reshape(NW, NCHUNK, CHUNK).astype(jnp.int32)
    out = _sc_lookup(idx, scale_bcast, weight)
    n, m = input.shape
    return out.reshape(m, n, DIM).transpose(1, 0, 2)
